# baseline (speedup 1.0000x reference)
import functools

import jax
import jax.numpy as jnp
from jax import lax
from jax.experimental import pallas as pl
from jax.experimental.pallas import tpu as pltpu
from jax.experimental.pallas import tpu_sc as plsc

N = 10000
E = 160000
B = 32
IN = 32
OUT = 64
NM = 5

NT = 16
NC = 2
LANES = 16
CH = 128
EPT = 10240
NCH = EPT // CH
REPT = E // NT
NPAD = 10240
RPT = NPAD // NT
LAST_ROWS = N - (NT - 1) * RPT
SPC = B // NC
NBLK = 2000


NBUF = 10
LAG = 5


def _sc_body(x0, rdst, cdst, rsrc, csrc, x1, x2, x3, x4,
             acc, deg1, deg2,
             dst_v, osrc, gbuf,
             acc_loc, prev_loc, inv1_loc, inv2_loc, ones_v,
             *sems):
  sem_g = sems[:NBUF]
  sem_s = sems[NBUF:2 * NBUF]
  sem_p = sems[2 * NBUF]
  c = lax.axis_index("c")
  t = lax.axis_index("s")
  r0 = pl.multiple_of(t * RPT, 8)
  is_last = t == NT - 1

  def fill_loop(ref, n16, val):
    def body(i, carry):
      ref[pl.ds(pl.multiple_of(i * LANES, 8), LANES)] = jnp.full(
          (LANES,), val, jnp.float32)
      return carry
    lax.fori_loop(0, n16, body, None)

  fill_loop(ones_v, CH // LANES, 1.0)
  fill_loop(inv1_loc, RPT // LANES, 0.0)

  pltpu.sync_copy(inv1_loc, deg1.at[pl.ds(r0, RPT)])
  pltpu.sync_copy(inv1_loc, deg2.at[pl.ds(r0, RPT)])
  plsc.subcore_barrier()

  def deg_pass(idx_hbm, deg_sh):
    pltpu.sync_copy(idx_hbm.at[t], dst_v)

    def deg_body(i, carry):
      pltpu.sync_copy(ones_v, deg_sh.at[dst_v.at[i]], add=True)
      return carry
    lax.fori_loop(0, NCH, deg_body, None)

  deg_pass(rdst, deg1)
  deg_pass(cdst, deg2)
  plsc.subcore_barrier()

  pltpu.sync_copy(deg1.at[pl.ds(r0, RPT)], inv1_loc)
  pltpu.sync_copy(deg2.at[pl.ds(r0, RPT)], inv2_loc)

  def inv_body(i, carry):
    o = pl.multiple_of(i * LANES, 8)
    for ref in (inv1_loc, inv2_loc):
      d = ref[pl.ds(o, LANES)]
      ref[pl.ds(o, LANES)] = jnp.where(d > 0.0, 1.0 / d, 0.0)
    return carry
  lax.fori_loop(0, RPT // LANES, inv_body, None)

  def spmm(gsrc, inv_loc, prev, out, soff):
    @plsc.parallel_loop(0, RPT, unroll=4)
    def _(i):
      for h in range(IN // LANES):
        acc_loc[i, pl.ds(h * LANES, LANES)] = jnp.zeros((LANES,), jnp.float32)
    pltpu.sync_copy(acc_loc, acc.at[pl.ds(r0, RPT)])
    plsc.subcore_barrier()

    row0 = soff + r0
    if prev is not None:
      @pl.when(jnp.logical_not(is_last))
      def _():
        pltpu.async_copy(prev.at[pl.ds(row0, RPT)], prev_loc, sem_p)

      @pl.when(is_last)
      def _():
        pltpu.async_copy(prev.at[pl.ds(row0, LAST_ROWS)],
                         prev_loc.at[pl.ds(0, LAST_ROWS)], sem_p)

    def start_gather(ci, bi):
      idx = osrc.at[pl.ds(pl.multiple_of(ci * CH, CH), CH)]
      pltpu.async_copy(gsrc.at[pl.ds(soff, N)].at[idx], gbuf.at[bi],
                       sem_g[bi])

    def wait_gather(bi):
      pltpu.make_async_copy(gsrc.at[pl.ds(0, CH)], gbuf.at[bi],
                            sem_g[bi]).wait()

    def start_scatter(ci, bi):
      pltpu.async_copy(gbuf.at[bi], acc.at[dst_v.at[ci]], sem_s[bi],
                       add=True)

    def wait_scatter(bi):
      pltpu.make_async_copy(gbuf.at[bi], acc.at[pl.ds(0, CH)],
                            sem_s[bi]).wait()

    for j in range(NBUF):
      start_gather(j, j)

    def ebody(g, carry):
      for j in range(NBUF):
        ci = g * NBUF + j
        wait_gather(j)
        start_scatter(ci, j)
        cd = ci - LAG
        jd = (j - LAG) % NBUF

        @pl.when(cd >= 0)
        def _():
          wait_scatter(jd)

          @pl.when(cd + NBUF < NCH)
          def _():
            start_gather(cd + NBUF, jd)
      return carry
    lax.fori_loop(0, NCH // NBUF, ebody, None)
    for j in range(NBUF - LAG, NBUF):
      wait_scatter(j)
    plsc.subcore_barrier()

    pltpu.sync_copy(acc.at[pl.ds(r0, RPT)], acc_loc)
    if prev is not None:
      @pl.when(jnp.logical_not(is_last))
      def _():
        pltpu.make_async_copy(prev.at[pl.ds(row0, RPT)], prev_loc,
                              sem_p).wait()

      @pl.when(is_last)
      def _():
        pltpu.make_async_copy(prev.at[pl.ds(row0, LAST_ROWS)],
                              prev_loc.at[pl.ds(0, LAST_ROWS)], sem_p).wait()

    if True:
      @plsc.parallel_loop(0, RPT, unroll=4)
      def _(r):
        iv = plsc.load_gather(inv_loc, [jnp.full((LANES,), r, jnp.int32)])
        if prev is not None:
          iv = iv + iv
        for h in range(IN // LANES):
          a = acc_loc[r, pl.ds(h * LANES, LANES)]
          o = iv * a
          if prev is not None:
            o = o - prev_loc[r, pl.ds(h * LANES, LANES)]
          acc_loc[r, pl.ds(h * LANES, LANES)] = o

    @pl.when(jnp.logical_not(is_last))
    def _():
      pltpu.sync_copy(acc_loc, out.at[pl.ds(row0, RPT)])

    @pl.when(is_last)
    def _():
      pltpu.sync_copy(acc_loc.at[pl.ds(0, LAST_ROWS)],
                      out.at[pl.ds(row0, LAST_ROWS)])

  def do_phase(src_hbm, dst_hbm, gsrc, inv_loc, prev, out):
    pltpu.sync_copy(src_hbm.at[t], osrc)
    pltpu.sync_copy(dst_hbm.at[t], dst_v)

    def sbody(sl, carry):
      soff = pl.multiple_of((c * SPC + sl) * N, 8)
      spmm(gsrc, inv_loc, prev, out, soff)
      return carry
    lax.fori_loop(0, SPC, sbody, None)

  do_phase(csrc, rdst, x0, inv1_loc, None, x1)
  do_phase(csrc, rdst, x1, inv1_loc, x0, x2)
  do_phase(rsrc, cdst, x1, inv2_loc, None, x3)
  do_phase(rsrc, cdst, x3, inv2_loc, x1, x4)


_sc_call = pl.kernel(
    _sc_body,
    out_type=[jax.ShapeDtypeStruct((B * N, IN), jnp.float32)] * 4,
    mesh=plsc.VectorSubcoreMesh(core_axis_name="c", subcore_axis_name="s"),
    scratch_types=[
        pltpu.VMEM_SHARED((NPAD, IN), jnp.float32),
        pltpu.VMEM_SHARED((NPAD,), jnp.float32),
        pltpu.VMEM_SHARED((NPAD,), jnp.float32),
        pltpu.VMEM((NCH, CH), jnp.int32),
        pltpu.VMEM((EPT,), jnp.int32),
        pltpu.VMEM((NBUF, CH, IN), jnp.float32),
        pltpu.VMEM((RPT, IN), jnp.float32),
        pltpu.VMEM((RPT, IN), jnp.float32),
        pltpu.VMEM((RPT,), jnp.float32),
        pltpu.VMEM((RPT,), jnp.float32),
        pltpu.VMEM((CH,), jnp.float32),
    ] + [pltpu.SemaphoreType.DMA] * (2 * NBUF + 1),
    compiler_params=pltpu.CompilerParams(
        needs_layout_passes=False, use_tc_tiling_on_sc=False),
)


def _final_body(x0r, x1r, x2r, x3r, x4r, wr, br, outr):
  acc = jnp.dot(x0r[0], wr[0], preferred_element_type=jnp.float32)
  for m, xr in enumerate((x1r, x2r, x3r, x4r)):
    acc = acc + jnp.dot(xr[0], wr[m + 1], preferred_element_type=jnp.float32)
  acc = acc + br[0][None, :]
  ys = [acc[:, q * OUT:(q + 1) * OUT] for q in range(4)]
  outr[0] = jnp.stack(ys, axis=1).reshape(N, OUT)


NR = B * N * IN // 128


@jax.jit
def kernel(inputs, edge_index, W, b):
  rows = edge_index[0].astype(jnp.int32)
  cols = edge_index[1].astype(jnp.int32)

  npad = EPT - REPT
  pad_src = jnp.arange(npad, dtype=jnp.int32) % 16
  pad_dst = pad_src + N

  def padded(a, padvals, shape):
    a2 = a.reshape(NT, REPT)
    p = jnp.broadcast_to(padvals, (NT, npad))
    return jnp.concatenate([a2, p], axis=1).reshape(shape)

  rdst = padded(rows, pad_dst, (NT, NCH, CH))
  cdst = padded(cols, pad_dst, (NT, NCH, CH))
  rsrc = padded(rows, pad_src, (NT, EPT))
  csrc = padded(cols, pad_src, (NT, EPT))

  x0f = inputs.reshape(B * N, IN)
  x1, x2, x3, x4 = _sc_call(x0f, rdst, cdst, rsrc, csrc)

  w5 = W.reshape(IN, NM, OUT).transpose(1, 0, 2)
  wblk = jnp.zeros((NM, 4 * IN, 4 * OUT), jnp.float32)
  for q in range(4):
    wblk = wblk.at[:, IN * q:IN * (q + 1), OUT * q:OUT * (q + 1)].set(w5)
  b4 = jnp.tile(b.reshape(-1), 4).reshape(1, 4 * OUT)
  xs = [x.reshape(B, N // 4, 4 * IN) for x in (x0f, x1, x2, x3, x4)]

  NRB = N // 4
  out = pl.pallas_call(
      _final_body,
      grid=(B,),
      in_specs=[pl.BlockSpec((1, NRB, 4 * IN), lambda bb: (bb, 0, 0))] * NM
      + [
          pl.BlockSpec((NM, 4 * IN, 4 * OUT), lambda bb: (0, 0, 0)),
          pl.BlockSpec((1, 4 * OUT), lambda bb: (0, 0)),
      ],
      out_specs=pl.BlockSpec((1, N, OUT), lambda bb: (bb, 0, 0)),
      out_shape=jax.ShapeDtypeStruct((B, N, OUT), jnp.float32),
  )(*xs, wblk, b4)
  return out


# speedup vs baseline: 1.0547x; 1.0547x over previous
import functools

import jax
import jax.numpy as jnp
from jax import lax
from jax.experimental import pallas as pl
from jax.experimental.pallas import tpu as pltpu
from jax.experimental.pallas import tpu_sc as plsc

N = 10000
E = 160000
B = 32
IN = 32
OUT = 64
NM = 5

NT = 16
NC = 2
LANES = 16
CH = 128
EPT = 10240
NCH = EPT // CH
REPT = E // NT
NPAD = 10240
RPT = NPAD // NT
LAST_ROWS = N - (NT - 1) * RPT
SPC = B // NC
NBLK = 2000


NBUF = 10
LAG = 3


def _sc_body(x0, rdst, cdst, rsrc, csrc, x1, x2, x3, x4,
             acc, deg1, deg2,
             dst_v, osrc, gbuf,
             acc_loc, prev_loc, inv1_loc, inv2_loc, ones_v,
             *sems):
  sem_g = sems[:NBUF]
  sem_s = sems[NBUF:2 * NBUF]
  sem_p = sems[2 * NBUF]
  c = lax.axis_index("c")
  t = lax.axis_index("s")
  r0 = pl.multiple_of(t * RPT, 8)
  is_last = t == NT - 1

  def fill_loop(ref, n16, val):
    def body(i, carry):
      ref[pl.ds(pl.multiple_of(i * LANES, 8), LANES)] = jnp.full(
          (LANES,), val, jnp.float32)
      return carry
    lax.fori_loop(0, n16, body, None)

  fill_loop(ones_v, CH // LANES, 1.0)
  fill_loop(inv1_loc, RPT // LANES, 0.0)

  pltpu.sync_copy(inv1_loc, deg1.at[pl.ds(r0, RPT)])
  pltpu.sync_copy(inv1_loc, deg2.at[pl.ds(r0, RPT)])
  plsc.subcore_barrier()

  def deg_pass(idx_hbm, deg_sh):
    pltpu.sync_copy(idx_hbm.at[t], dst_v)

    def deg_body(i, carry):
      pltpu.sync_copy(ones_v, deg_sh.at[dst_v.at[i]], add=True)
      return carry
    lax.fori_loop(0, NCH, deg_body, None)

  deg_pass(rdst, deg1)
  deg_pass(cdst, deg2)
  plsc.subcore_barrier()

  pltpu.sync_copy(deg1.at[pl.ds(r0, RPT)], inv1_loc)
  pltpu.sync_copy(deg2.at[pl.ds(r0, RPT)], inv2_loc)

  def inv_body(i, carry):
    o = pl.multiple_of(i * LANES, 8)
    for ref in (inv1_loc, inv2_loc):
      d = ref[pl.ds(o, LANES)]
      ref[pl.ds(o, LANES)] = jnp.where(d > 0.0, 1.0 / d, 0.0)
    return carry
  lax.fori_loop(0, RPT // LANES, inv_body, None)

  def spmm(gsrc, inv_loc, prev, out, soff):
    @plsc.parallel_loop(0, RPT, unroll=4)
    def _(i):
      for h in range(IN // LANES):
        acc_loc[i, pl.ds(h * LANES, LANES)] = jnp.zeros((LANES,), jnp.float32)
    pltpu.sync_copy(acc_loc, acc.at[pl.ds(r0, RPT)])
    plsc.subcore_barrier()

    row0 = soff + r0
    if prev is not None:
      @pl.when(jnp.logical_not(is_last))
      def _():
        pltpu.async_copy(prev.at[pl.ds(row0, RPT)], prev_loc, sem_p)

      @pl.when(is_last)
      def _():
        pltpu.async_copy(prev.at[pl.ds(row0, LAST_ROWS)],
                         prev_loc.at[pl.ds(0, LAST_ROWS)], sem_p)

    def start_gather(ci, bi):
      pltpu.async_copy(gsrc.at[pl.ds(soff, N)].at[osrc.at[ci]], gbuf.at[bi],
                       sem_g[bi])

    def wait_gather(bi):
      pltpu.make_async_copy(gsrc.at[pl.ds(0, CH)], gbuf.at[bi],
                            sem_g[bi]).wait()

    def start_scatter(ci, bi):
      pltpu.async_copy(gbuf.at[bi], acc.at[dst_v.at[ci]], sem_s[bi],
                       add=True)

    def wait_scatter(bi):
      pltpu.make_async_copy(gbuf.at[bi], acc.at[pl.ds(0, CH)],
                            sem_s[bi]).wait()

    for j in range(NBUF):
      start_gather(j, j)

    def ebody(g, carry):
      for j in range(NBUF):
        ci = g * NBUF + j
        wait_gather(j)
        start_scatter(ci, j)
        cd = ci - LAG
        jd = (j - LAG) % NBUF

        @pl.when(cd >= 0)
        def _():
          wait_scatter(jd)

          @pl.when(cd + NBUF < NCH)
          def _():
            start_gather(cd + NBUF, jd)
      return carry
    lax.fori_loop(0, NCH // NBUF, ebody, None)
    for j in range(NBUF - LAG, NBUF):
      wait_scatter(j)
    plsc.subcore_barrier()

    pltpu.sync_copy(acc.at[pl.ds(r0, RPT)], acc_loc)
    if prev is not None:
      @pl.when(jnp.logical_not(is_last))
      def _():
        pltpu.make_async_copy(prev.at[pl.ds(row0, RPT)], prev_loc,
                              sem_p).wait()

      @pl.when(is_last)
      def _():
        pltpu.make_async_copy(prev.at[pl.ds(row0, LAST_ROWS)],
                              prev_loc.at[pl.ds(0, LAST_ROWS)], sem_p).wait()

    if True:
      @plsc.parallel_loop(0, RPT, unroll=4)
      def _(r):
        iv = plsc.load_gather(inv_loc, [jnp.full((LANES,), r, jnp.int32)])
        if prev is not None:
          iv = iv + iv
        for h in range(IN // LANES):
          a = acc_loc[r, pl.ds(h * LANES, LANES)]
          o = iv * a
          if prev is not None:
            o = o - prev_loc[r, pl.ds(h * LANES, LANES)]
          acc_loc[r, pl.ds(h * LANES, LANES)] = o

    @pl.when(jnp.logical_not(is_last))
    def _():
      pltpu.sync_copy(acc_loc, out.at[pl.ds(row0, RPT)])

    @pl.when(is_last)
    def _():
      pltpu.sync_copy(acc_loc.at[pl.ds(0, LAST_ROWS)],
                      out.at[pl.ds(row0, LAST_ROWS)])

  def do_phase(src_hbm, dst_hbm, gsrc, inv_loc, prev, out):
    pltpu.sync_copy(src_hbm.at[t], osrc)
    pltpu.sync_copy(dst_hbm.at[t], dst_v)

    def sbody(sl, carry):
      soff = pl.multiple_of((c * SPC + sl) * N, 8)
      spmm(gsrc, inv_loc, prev, out, soff)
      return carry
    lax.fori_loop(0, SPC, sbody, None)

  do_phase(csrc, rdst, x0, inv1_loc, None, x1)
  do_phase(csrc, rdst, x1, inv1_loc, x0, x2)
  do_phase(rsrc, cdst, x1, inv2_loc, None, x3)
  do_phase(rsrc, cdst, x3, inv2_loc, x1, x4)


_sc_call = pl.kernel(
    _sc_body,
    out_type=[jax.ShapeDtypeStruct((B * N, IN), jnp.float32)] * 4,
    mesh=plsc.VectorSubcoreMesh(core_axis_name="c", subcore_axis_name="s"),
    scratch_types=[
        pltpu.VMEM_SHARED((NPAD, IN), jnp.float32),
        pltpu.VMEM_SHARED((NPAD,), jnp.float32),
        pltpu.VMEM_SHARED((NPAD,), jnp.float32),
        pltpu.VMEM((NCH, CH), jnp.int32),
        pltpu.VMEM((NCH, CH), jnp.int32),
        pltpu.VMEM((NBUF, CH, IN), jnp.float32),
        pltpu.VMEM((RPT, IN), jnp.float32),
        pltpu.VMEM((RPT, IN), jnp.float32),
        pltpu.VMEM((RPT,), jnp.float32),
        pltpu.VMEM((RPT,), jnp.float32),
        pltpu.VMEM((CH,), jnp.float32),
    ] + [pltpu.SemaphoreType.DMA] * (2 * NBUF + 1),
    compiler_params=pltpu.CompilerParams(
        needs_layout_passes=False, use_tc_tiling_on_sc=False),
)


def _final_body(x0r, x1r, x2r, x3r, x4r, wr, br, outr):
  acc = jnp.dot(x0r[0], wr[0], preferred_element_type=jnp.float32)
  for m, xr in enumerate((x1r, x2r, x3r, x4r)):
    acc = acc + jnp.dot(xr[0], wr[m + 1], preferred_element_type=jnp.float32)
  acc = acc + br[0][None, :]
  ys = [acc[:, q * OUT:(q + 1) * OUT] for q in range(4)]
  outr[0] = jnp.stack(ys, axis=1).reshape(N, OUT)


NR = B * N * IN // 128


@jax.jit
def kernel(inputs, edge_index, W, b):
  rows = edge_index[0].astype(jnp.int32)
  cols = edge_index[1].astype(jnp.int32)

  npad = EPT - REPT
  pad_src = jnp.arange(npad, dtype=jnp.int32) % 16
  pad_dst = pad_src + N

  def padded(a, padvals, shape):
    a2 = a.reshape(NT, REPT)
    p = jnp.broadcast_to(padvals, (NT, npad))
    return jnp.concatenate([a2, p], axis=1).reshape(shape)

  rdst = padded(rows, pad_dst, (NT, NCH, CH))
  cdst = padded(cols, pad_dst, (NT, NCH, CH))
  rsrc = padded(rows, pad_src, (NT, NCH, CH))
  csrc = padded(cols, pad_src, (NT, NCH, CH))

  x0f = inputs.reshape(B * N, IN)
  x1, x2, x3, x4 = _sc_call(x0f, rdst, cdst, rsrc, csrc)

  w5 = W.reshape(IN, NM, OUT).transpose(1, 0, 2)
  wblk = jnp.zeros((NM, 4 * IN, 4 * OUT), jnp.float32)
  for q in range(4):
    wblk = wblk.at[:, IN * q:IN * (q + 1), OUT * q:OUT * (q + 1)].set(w5)
  b4 = jnp.tile(b.reshape(-1), 4).reshape(1, 4 * OUT)
  xs = [x.reshape(B, N // 4, 4 * IN) for x in (x0f, x1, x2, x3, x4)]

  NRB = N // 4
  out = pl.pallas_call(
      _final_body,
      grid=(B,),
      in_specs=[pl.BlockSpec((1, NRB, 4 * IN), lambda bb: (bb, 0, 0))] * NM
      + [
          pl.BlockSpec((NM, 4 * IN, 4 * OUT), lambda bb: (0, 0, 0)),
          pl.BlockSpec((1, 4 * OUT), lambda bb: (0, 0)),
      ],
      out_specs=pl.BlockSpec((1, N, OUT), lambda bb: (bb, 0, 0)),
      out_shape=jax.ShapeDtypeStruct((B, N, OUT), jnp.float32),
  )(*xs, wblk, b4)
  return out
